# baseline (speedup 1.0000x reference)
import functools

import jax
import jax.numpy as jnp
from jax import lax
from jax.experimental import pallas as pl
from jax.experimental.pallas import tpu as pltpu
from jax.experimental.pallas import tpu_sc as plsc

DIM = 64
B = 4096 * 200
NC, NS = 2, 16
NW = NC * NS
B_PER_W = B // NW
IDXW = 128
CHUNK = 1024
SUB = CHUNK // IDXW
NCHUNK = B_PER_W // CHUNK

_mesh = plsc.VectorSubcoreMesh(core_axis_name="c", subcore_axis_name="s")


@functools.partial(
    pl.kernel,
    mesh=_mesh,
    compiler_params=pltpu.CompilerParams(use_tc_tiling_on_sc=False),
    out_type=jax.ShapeDtypeStruct((B, DIM), jnp.float32),
    scratch_types=[
        pltpu.VMEM((SUB, IDXW), jnp.int32),
        pltpu.VMEM((CHUNK, DIM), jnp.float32),
        pltpu.SemaphoreType.DMA,
    ],
)
def _gather(idx_hbm, table_hbm, out_hbm, idx_v, rows_v, sem):
    wid = lax.axis_index("s") * NC + lax.axis_index("c")
    base = wid * B_PER_W

    def body(i, carry):
        off = pl.multiple_of(base + i * CHUNK, CHUNK)
        idx_row = pl.multiple_of(base // IDXW + i * SUB, SUB)
        pltpu.sync_copy(idx_hbm.at[pl.ds(idx_row, SUB)], idx_v)
        copies = []
        for j in range(SUB):
            copies.append(
                pltpu.async_copy(
                    table_hbm.at[idx_v.at[j]],
                    rows_v.at[pl.ds(j * IDXW, IDXW)],
                    sem,
                )
            )
        for c in copies:
            c.wait()
        pltpu.sync_copy(rows_v, out_hbm.at[pl.ds(off, CHUNK)])
        return carry

    lax.fori_loop(0, NCHUNK, body, 0)


def kernel(x, weight):
    idx = x.reshape(B // IDXW, IDXW).astype(jnp.int32)
    out = _gather(idx, weight)
    return out.reshape(x.shape[0], x.shape[1], DIM)


# speedup vs baseline: 1.0175x; 1.0175x over previous
import functools

import jax
import jax.numpy as jnp
from jax import lax
from jax.experimental import pallas as pl
from jax.experimental.pallas import tpu as pltpu
from jax.experimental.pallas import tpu_sc as plsc

DIM = 64
B = 4096 * 200
NC, NS = 2, 16
NW = NC * NS
B_PER_W = B // NW
IDXW = 128
CHUNK = 256
SUB = CHUNK // IDXW
NCHUNK = B_PER_W // CHUNK
NBUF = 4
IDX_ROWS = B_PER_W // IDXW

_mesh = plsc.VectorSubcoreMesh(core_axis_name="c", subcore_axis_name="s")


@functools.partial(
    pl.kernel,
    mesh=_mesh,
    compiler_params=pltpu.CompilerParams(use_tc_tiling_on_sc=False),
    out_type=jax.ShapeDtypeStruct((B, DIM), jnp.float32),
    scratch_types=[
        pltpu.VMEM((IDX_ROWS, IDXW), jnp.int32),
        pltpu.VMEM((NBUF, CHUNK, DIM), jnp.float32),
        pltpu.SemaphoreType.DMA((NBUF,)),
        pltpu.SemaphoreType.DMA((NBUF,)),
    ],
)
def _gather(idx_hbm, table_hbm, out_hbm, idx_v, rows_v, gsem, ssem):
    wid = lax.axis_index("s") * NC + lax.axis_index("c")
    base = wid * B_PER_W

    idx_row0 = pl.multiple_of(wid * IDX_ROWS, 8)
    pltpu.sync_copy(idx_hbm.at[pl.ds(idx_row0, IDX_ROWS)], idx_v)

    def start_gather(c, b):
        for j in range(SUB):
            pltpu.async_copy(
                table_hbm.at[idx_v.at[c * SUB + j]],
                rows_v.at[b, pl.ds(j * IDXW, IDXW)],
                gsem.at[b],
            )

    def wait_gather(b):
        for j in range(SUB):
            pltpu.make_async_copy(
                table_hbm.at[idx_v.at[j]],
                rows_v.at[b, pl.ds(j * IDXW, IDXW)],
                gsem.at[b],
            ).wait()

    def start_store(c, b):
        off = pl.multiple_of(base + c * CHUNK, CHUNK)
        pltpu.async_copy(rows_v.at[b], out_hbm.at[pl.ds(off, CHUNK)], ssem.at[b])

    def wait_store(b):
        pltpu.make_async_copy(
            rows_v.at[b], out_hbm.at[pl.ds(base, CHUNK)], ssem.at[b]
        ).wait()

    for b in range(NBUF):
        start_gather(b, b)

    @pl.loop(0, NCHUNK, step=NBUF)
    def _ring(g):
        for b in range(NBUF):
            c = g + b
            wait_gather(b)
            start_store(c, b)
            nxt = c + NBUF

            @pl.when(nxt < NCHUNK)
            def _():
                wait_store(b)
                start_gather(nxt, b)

    for b in range(NBUF):
        wait_store(b)


def kernel(x, weight):
    idx = x.reshape(B // IDXW, IDXW).astype(jnp.int32)
    out = _gather(idx, weight)
    return out.reshape(x.shape[0], x.shape[1], DIM)


# speedup vs baseline: 1.0214x; 1.0038x over previous
import functools

import jax
import jax.numpy as jnp
from jax import lax
from jax.experimental import pallas as pl
from jax.experimental.pallas import tpu as pltpu
from jax.experimental.pallas import tpu_sc as plsc

DIM = 64
B = 4096 * 200
NC, NS = 2, 16
NW = NC * NS
B_PER_W = B // NW
IDXW = 128
CHUNK = 256
SUB = CHUNK // IDXW
NCHUNK = B_PER_W // CHUNK
NBUF = 4
IDX_ROWS = B_PER_W // IDXW

_mesh = plsc.VectorSubcoreMesh(core_axis_name="c", subcore_axis_name="s")


@functools.partial(
    pl.kernel,
    mesh=_mesh,
    compiler_params=pltpu.CompilerParams(use_tc_tiling_on_sc=False),
    out_type=jax.ShapeDtypeStruct((B, DIM), jnp.float32),
    scratch_types=[
        pltpu.VMEM((IDX_ROWS, IDXW), jnp.int32),
        pltpu.VMEM((NBUF, CHUNK, DIM), jnp.float32),
        pltpu.SemaphoreType.DMA((NBUF,)),
        pltpu.SemaphoreType.DMA((NBUF,)),
    ],
)
def _gather(idx_hbm, table_hbm, out_hbm, idx_v, rows_v, gsem, ssem):
    wid = lax.axis_index("s") * NC + lax.axis_index("c")
    base = wid * B_PER_W

    idx_row0 = pl.multiple_of(wid * IDX_ROWS, 8)
    pltpu.sync_copy(idx_hbm.at[pl.ds(idx_row0, IDX_ROWS)], idx_v)

    def start_gather(c, b):
        for j in range(SUB):
            pltpu.async_copy(
                table_hbm.at[idx_v.at[c * SUB + j]],
                rows_v.at[b, pl.ds(j * IDXW, IDXW)],
                gsem.at[b],
            )

    def wait_gather(b):
        for j in range(SUB):
            pltpu.make_async_copy(
                table_hbm.at[idx_v.at[j]],
                rows_v.at[b, pl.ds(j * IDXW, IDXW)],
                gsem.at[b],
            ).wait()

    def start_store(c, b):
        off = pl.multiple_of(base + c * CHUNK, CHUNK)
        pltpu.async_copy(rows_v.at[b], out_hbm.at[pl.ds(off, CHUNK)], ssem.at[b])

    def wait_store(b):
        pltpu.make_async_copy(
            rows_v.at[b], out_hbm.at[pl.ds(base, CHUNK)], ssem.at[b]
        ).wait()

    for b in range(NBUF):
        start_gather(b, b)

    @pl.loop(0, NCHUNK, step=NBUF)
    def _ring(g):
        for b in range(NBUF):
            c = g + b
            wait_gather(b)
            start_store(c, b)
            nxt = c + NBUF

            @pl.when(nxt < NCHUNK)
            def _():
                wait_store(b)
                start_gather(nxt, b)

    for b in range(NBUF):
        wait_store(b)


def kernel(x, weight):
    idx = x.reshape(B // IDXW, IDXW).astype(jnp.int32)
    wlin = weight.reshape(-1).reshape(weight.shape)
    out = _gather(idx, wlin)
    return out.reshape(-1).reshape(x.shape[0], x.shape[1], DIM)
